# baseline (speedup 1.0000x reference)
import functools

import jax
import jax.numpy as jnp
from jax import lax
from jax.experimental import pallas as pl
from jax.experimental.pallas import tpu as pltpu
from jax.experimental.pallas import tpu_sc as plsc

B = 4
T = 64
BT = B * T
P_HEMI = 320
V = 153
H = 40962
VT = 2 * H
PI_FLAT = P_HEMI * V
ROWS_TOTAL = 2 * PI_FLAT

NUM_TILES = 32
CHUNK = 96
NCHUNKS = ROWS_TOTAL // CHUNK
CHUNKS_PER_HEMI = PI_FLAT // CHUNK
LANES = 16

_mesh = plsc.VectorSubcoreMesh(core_axis_name="c", subcore_axis_name="s")


@functools.partial(
    pl.kernel,
    out_type=jax.ShapeDtypeStruct((ROWS_TOTAL, BT), jnp.float32),
    mesh=_mesh,
    scratch_types=[
        pltpu.VMEM((CHUNK,), jnp.int32),
        pltpu.VMEM((CHUNK,), jnp.int32),
        pltpu.VMEM((CHUNK, BT), jnp.float32),
        pltpu.VMEM((CHUNK, BT), jnp.float32),
        pltpu.SemaphoreType.DMA,
        pltpu.SemaphoreType.DMA,
        pltpu.SemaphoreType.DMA,
        pltpu.SemaphoreType.DMA,
    ],
)
def _sc_gather(xg_hbm, pi_hbm, out_hbm, idx0, idx1, buf0, buf1,
               semg0, semg1, semo0, semo1):
    wid = lax.axis_index("s") * 2 + lax.axis_index("c")
    nchunks_w = jnp.where(wid < NCHUNKS % NUM_TILES, NCHUNKS // NUM_TILES + 1,
                          NCHUNKS // NUM_TILES)
    idxs = (idx0, idx1)
    bufs = (buf0, buf1)
    semgs = (semg0, semg1)
    semos = (semo0, semo1)

    def load_idx(i, idx_v):
        c = wid + i * NUM_TILES
        j0 = jnp.where(c < CHUNKS_PER_HEMI, c, c - CHUNKS_PER_HEMI) * CHUNK
        off = jnp.where(c < CHUNKS_PER_HEMI, 0, H).astype(jnp.int32)
        pltpu.sync_copy(pi_hbm.at[pl.ds(j0, CHUNK)], idx_v)
        off_vec = jnp.broadcast_to(off, (LANES,))
        for k in range(CHUNK // LANES):
            sl = pl.ds(k * LANES, LANES)
            idx_v[sl] = idx_v[sl] + off_vec

    load_idx(0, idx0)
    pltpu.async_copy(xg_hbm.at[idx0], buf0, semg0)

    def chunk_body(i, carry):
        c = wid + i * NUM_TILES
        for par in range(2):
            @pl.when(i % 2 == par)
            def _():
                @pl.when(i + 1 < nchunks_w)
                def _():
                    load_idx(i + 1, idxs[1 - par])

                @pl.when(i >= 1)
                def _():
                    pltpu.make_async_copy(
                        bufs[1 - par], out_hbm.at[pl.ds(0, CHUNK)],
                        semos[1 - par]).wait()

                pltpu.make_async_copy(
                    xg_hbm.at[idxs[par]], bufs[par], semgs[par]).wait()

                @pl.when(i + 1 < nchunks_w)
                def _():
                    pltpu.async_copy(
                        xg_hbm.at[idxs[1 - par]], bufs[1 - par],
                        semgs[1 - par])

                pltpu.async_copy(bufs[par],
                                 out_hbm.at[pl.ds(c * CHUNK, CHUNK)],
                                 semos[par])
        return carry

    lax.fori_loop(0, nchunks_w, chunk_body, 0)

    for par in range(2):
        @pl.when((nchunks_w - 1) % 2 == par)
        def _():
            pltpu.make_async_copy(bufs[par], out_hbm.at[pl.ds(0, CHUNK)],
                                  semos[par]).wait()


_HB = 4096


def _repack_body(in_ref, out_ref):
    out_ref[...] = jnp.concatenate([in_ref[b] for b in range(B)], axis=1)


def _stage_a(x):
    return pl.pallas_call(
        _repack_body,
        grid=(pl.cdiv(VT, _HB),),
        in_specs=[pl.BlockSpec((B, _HB, T), lambda h: (0, h, 0))],
        out_specs=pl.BlockSpec((_HB, BT), lambda h: (h, 0)),
        out_shape=jax.ShapeDtypeStruct((VT, BT), jnp.float32),
    )(x)


_PB = 16


def _unpack_body(in_ref, out_ref):
    data_t = in_ref[...].T
    for b in range(B):
        out_ref[b] = data_t[b * T:(b + 1) * T].reshape(T, _PB, V)


def _stage_c(g):
    return pl.pallas_call(
        _unpack_body,
        grid=((2 * P_HEMI) // _PB,),
        in_specs=[pl.BlockSpec((_PB * V, BT), lambda p: (p, 0))],
        out_specs=pl.BlockSpec((B, T, _PB, V), lambda p: (0, 0, p, 0)),
        out_shape=jax.ShapeDtypeStruct((B, T, 2 * P_HEMI, V), jnp.float32),
    )(g)


def kernel(x, patch_indices):
    xg = _stage_a(x)
    g = _sc_gather(xg, patch_indices.reshape(PI_FLAT))
    return _stage_c(g)


# speedup vs baseline: 2.1421x; 2.1421x over previous
import functools

import jax
import jax.numpy as jnp
from jax import lax
from jax.experimental import pallas as pl
from jax.experimental.pallas import tpu as pltpu
from jax.experimental.pallas import tpu_sc as plsc

B = 4
T = 64
BT = B * T
P_HEMI = 320
P2 = 2 * P_HEMI
V = 153
H = 40962
VT = 2 * H
ROWS_TOTAL = V * P2

NUM_TILES = 32
CHUNK = 64
NCHUNKS = ROWS_TOTAL // CHUNK
CPV = P2 // CHUNK
LANES = 16

_mesh = plsc.VectorSubcoreMesh(core_axis_name="c", subcore_axis_name="s")


@functools.partial(
    pl.kernel,
    out_type=jax.ShapeDtypeStruct((ROWS_TOTAL, BT), jnp.float32),
    mesh=_mesh,
    scratch_types=[
        pltpu.VMEM((CHUNK,), jnp.int32),
        pltpu.VMEM((CHUNK,), jnp.int32),
        pltpu.VMEM((CHUNK, BT), jnp.float32),
        pltpu.VMEM((CHUNK, BT), jnp.float32),
        pltpu.SemaphoreType.DMA,
        pltpu.SemaphoreType.DMA,
        pltpu.SemaphoreType.DMA,
        pltpu.SemaphoreType.DMA,
    ],
)
def _sc_gather(xg_hbm, pi2_hbm, out_hbm, idx0, idx1, buf0, buf1,
               semg0, semg1, semo0, semo1):
    wid = lax.axis_index("s") * 2 + lax.axis_index("c")
    nchunks_w = jnp.where(wid < NCHUNKS % NUM_TILES, NCHUNKS // NUM_TILES + 1,
                          NCHUNKS // NUM_TILES)
    idxs = (idx0, idx1)
    bufs = (buf0, buf1)
    semgs = (semg0, semg1)
    semos = (semo0, semo1)

    def load_idx(i, idx_v):
        c = wid + i * NUM_TILES
        off = jnp.where(lax.rem(c, CPV) < CPV // 2, 0, H).astype(jnp.int32)
        pltpu.sync_copy(pi2_hbm.at[pl.ds(c * CHUNK, CHUNK)], idx_v)
        off_vec = jnp.broadcast_to(off, (LANES,))
        for k in range(CHUNK // LANES):
            sl = pl.ds(k * LANES, LANES)
            idx_v[sl] = idx_v[sl] + off_vec

    load_idx(0, idx0)
    pltpu.async_copy(xg_hbm.at[idx0], buf0, semg0)

    def chunk_body(i, carry):
        c = wid + i * NUM_TILES
        for par in range(2):
            @pl.when(i % 2 == par)
            def _():
                @pl.when(i + 1 < nchunks_w)
                def _():
                    load_idx(i + 1, idxs[1 - par])

                @pl.when(i >= 1)
                def _():
                    pltpu.make_async_copy(
                        bufs[1 - par], out_hbm.at[pl.ds(0, CHUNK)],
                        semos[1 - par]).wait()

                pltpu.make_async_copy(
                    xg_hbm.at[idxs[par]], bufs[par], semgs[par]).wait()

                @pl.when(i + 1 < nchunks_w)
                def _():
                    pltpu.async_copy(
                        xg_hbm.at[idxs[1 - par]], bufs[1 - par],
                        semgs[1 - par])

                pltpu.async_copy(bufs[par],
                                 out_hbm.at[pl.ds(c * CHUNK, CHUNK)],
                                 semos[par])
        return carry

    lax.fori_loop(0, nchunks_w, chunk_body, 0)

    for par in range(2):
        @pl.when((nchunks_w - 1) % 2 == par)
        def _():
            pltpu.make_async_copy(bufs[par], out_hbm.at[pl.ds(0, CHUNK)],
                                  semos[par]).wait()


_HB = 4096


def _repack_body(in_ref, out_ref):
    out_ref[...] = jnp.concatenate([in_ref[b].T for b in range(B)], axis=1)


def _stage_a(xt):
    return pl.pallas_call(
        _repack_body,
        grid=(pl.cdiv(VT, _HB),),
        in_specs=[pl.BlockSpec((B, T, _HB), lambda h: (0, 0, h))],
        out_specs=pl.BlockSpec((_HB, BT), lambda h: (h, 0)),
        out_shape=jax.ShapeDtypeStruct((VT, BT), jnp.float32),
    )(xt)


_VB = 9


def _unpack_body(in_ref, out_ref):
    data_t = in_ref[...].T
    for b in range(B):
        for vi in range(_VB):
            out_ref[b, vi] = data_t[b * T:(b + 1) * T,
                                    vi * P2:(vi + 1) * P2]


def _stage_c(g):
    return pl.pallas_call(
        _unpack_body,
        grid=(V // _VB,),
        in_specs=[pl.BlockSpec((_VB * P2, BT), lambda v: (v, 0))],
        out_specs=pl.BlockSpec((B, _VB, T, P2), lambda v: (0, v, 0, 0)),
        out_shape=jax.ShapeDtypeStruct((B, V, T, P2), jnp.float32),
    )(g)


def kernel(x, patch_indices):
    xt = jnp.transpose(x, (0, 2, 1))
    piT = patch_indices.T
    pi2 = jnp.concatenate([piT, piT], axis=1).reshape(ROWS_TOTAL)
    xg = _stage_a(xt)
    g = _sc_gather(xg, pi2)
    out_perm = _stage_c(g)
    return jnp.transpose(out_perm, (0, 2, 3, 1))


# speedup vs baseline: 2.2806x; 1.0647x over previous
import functools

import jax
import jax.numpy as jnp
from jax import lax
from jax.experimental import pallas as pl
from jax.experimental.pallas import tpu as pltpu
from jax.experimental.pallas import tpu_sc as plsc

B = 4
T = 64
BT = B * T
P_HEMI = 320
P2 = 2 * P_HEMI
V = 153
H = 40962
VT = 2 * H
ROWS_TOTAL = V * P2

NUM_TILES = 32
CHUNK = 128
NCHUNKS = ROWS_TOTAL // CHUNK
LANES = 16

_mesh = plsc.VectorSubcoreMesh(core_axis_name="c", subcore_axis_name="s")


@functools.partial(
    pl.kernel,
    out_type=jax.ShapeDtypeStruct((ROWS_TOTAL, BT), jnp.float32),
    mesh=_mesh,
    scratch_types=[
        pltpu.VMEM((CHUNK,), jnp.int32),
        pltpu.VMEM((CHUNK,), jnp.int32),
        pltpu.VMEM((CHUNK, BT), jnp.float32),
        pltpu.VMEM((CHUNK, BT), jnp.float32),
        pltpu.SemaphoreType.DMA,
        pltpu.SemaphoreType.DMA,
        pltpu.SemaphoreType.DMA,
        pltpu.SemaphoreType.DMA,
    ],
)
def _sc_gather(xg_hbm, pi2_hbm, out_hbm, idx0, idx1, buf0, buf1,
               semg0, semg1, semo0, semo1):
    wid = lax.axis_index("s") * 2 + lax.axis_index("c")
    nchunks_w = jnp.where(wid < NCHUNKS % NUM_TILES, NCHUNKS // NUM_TILES + 1,
                          NCHUNKS // NUM_TILES)
    idxs = (idx0, idx1)
    bufs = (buf0, buf1)
    semgs = (semg0, semg1)
    semos = (semo0, semo1)

    def load_idx(i, idx_v):
        c = wid + i * NUM_TILES
        pltpu.sync_copy(pi2_hbm.at[pl.ds(c * CHUNK, CHUNK)], idx_v)

    load_idx(0, idx0)
    pltpu.async_copy(xg_hbm.at[idx0], buf0, semg0)

    def chunk_body(i, carry):
        c = wid + i * NUM_TILES
        for par in range(2):
            @pl.when(i % 2 == par)
            def _():
                @pl.when(i + 1 < nchunks_w)
                def _():
                    load_idx(i + 1, idxs[1 - par])

                @pl.when(i >= 1)
                def _():
                    pltpu.make_async_copy(
                        bufs[1 - par], out_hbm.at[pl.ds(0, CHUNK)],
                        semos[1 - par]).wait()

                pltpu.make_async_copy(
                    xg_hbm.at[idxs[par]], bufs[par], semgs[par]).wait()

                @pl.when(i + 1 < nchunks_w)
                def _():
                    pltpu.async_copy(
                        xg_hbm.at[idxs[1 - par]], bufs[1 - par],
                        semgs[1 - par])

                pltpu.async_copy(bufs[par],
                                 out_hbm.at[pl.ds(c * CHUNK, CHUNK)],
                                 semos[par])
        return carry

    lax.fori_loop(0, nchunks_w, chunk_body, 0)

    for par in range(2):
        @pl.when((nchunks_w - 1) % 2 == par)
        def _():
            pltpu.make_async_copy(bufs[par], out_hbm.at[pl.ds(0, CHUNK)],
                                  semos[par]).wait()


_HB = 4096


def _repack_body(in_ref, out_ref):
    out_ref[...] = jnp.concatenate([in_ref[b].T for b in range(B)], axis=1)


def _stage_a(xt):
    return pl.pallas_call(
        _repack_body,
        grid=(pl.cdiv(VT, _HB),),
        in_specs=[pl.BlockSpec((B, T, _HB), lambda h: (0, 0, h))],
        out_specs=pl.BlockSpec((_HB, BT), lambda h: (h, 0)),
        out_shape=jax.ShapeDtypeStruct((VT, BT), jnp.float32),
    )(xt)


_VB = 9


def _unpack_body(in_ref, out_ref):
    data_t = in_ref[...].T
    for b in range(B):
        for vi in range(_VB):
            out_ref[b, vi] = data_t[b * T:(b + 1) * T,
                                    vi * P2:(vi + 1) * P2]


def _stage_c(g):
    return pl.pallas_call(
        _unpack_body,
        grid=(V // _VB,),
        in_specs=[pl.BlockSpec((_VB * P2, BT), lambda v: (v, 0))],
        out_specs=pl.BlockSpec((B, _VB, T, P2), lambda v: (0, v, 0, 0)),
        out_shape=jax.ShapeDtypeStruct((B, V, T, P2), jnp.float32),
    )(g)


def kernel(x, patch_indices):
    xt = jnp.transpose(x, (0, 2, 1))
    piT = patch_indices.T
    pi2 = jnp.concatenate([piT, piT + H], axis=1).reshape(ROWS_TOTAL)
    xg = _stage_a(xt)
    g = _sc_gather(xg, pi2)
    out_perm = _stage_c(g)
    return jnp.transpose(out_perm, (0, 2, 3, 1))
